# Initial kernel scaffold; baseline (speedup 1.0000x reference)
import jax
import jax.numpy as jnp
from jax.experimental import pallas as pl


def kernel(x, edge_index, W1_l, W1_r, b1, W2_l, W2_r, b2):
    raise NotImplementedError("write your pallas kernel here")


# speedup vs baseline: 20.9742x; 20.9742x over previous
import functools

import jax
import jax.numpy as jnp
from jax import lax
from jax.experimental import pallas as pl
from jax.experimental.pallas import tpu as pltpu
from jax.experimental.pallas import tpu_sc as plsc

N = 10000
E = 320000
IN_CH = 128
HID = 16

NC, NS = 2, 16
NW = NC * NS
EPW = E // NW

CHUNK = 2048
NCHUNK = 5
EPW_PAD = NCHUNK * CHUNK

NPAD = 10112
ROWS_PER_TILE = NPAD // NS


def _tc_transform(x_ref, wl_ref, wr_ref, y1_ref, xr_ref):
    xx = x_ref[...]
    y1_ref[...] = lax.dot(xx, wl_ref[...], precision=lax.Precision.HIGHEST,
                          preferred_element_type=jnp.float32)
    xr_ref[...] = lax.dot(xx, wr_ref[...], precision=lax.Precision.HIGHEST,
                          preferred_element_type=jnp.float32)


_transform_call = pl.pallas_call(
    _tc_transform,
    out_shape=(jax.ShapeDtypeStruct((N, HID), jnp.float32),
               jax.ShapeDtypeStruct((N, HID), jnp.float32)),
)


def _sc_layer1(y1_hbm, srcf_hbm, dstf_hbm, agg_out, cnt_out,
               src_c, dst_c, rows_v, zrow_v, cnt_v, acc_sh):
    cid = lax.axis_index("c")
    sid = lax.axis_index("s")
    wid = cid * NS + sid

    zeros16 = jnp.zeros((16,), jnp.float32)

    def zb(i, carry):
        zrow_v[i, :] = zeros16
        cnt_v[pl.ds(i * 16, 16)] = zeros16
        return carry

    lax.fori_loop(0, ROWS_PER_TILE, zb, 0)
    pltpu.sync_copy(zrow_v, acc_sh.at[pl.ds(sid * ROWS_PER_TILE, ROWS_PER_TILE), :])

    plsc.subcore_barrier()

    ones16 = jnp.full((16,), 1.0, jnp.float32)
    vregs_per_chunk = CHUNK // 16

    def chunk(j, carry):
        base = wid * EPW_PAD + j * CHUNK
        pltpu.sync_copy(srcf_hbm.at[pl.ds(base, CHUNK)], src_c)
        pltpu.sync_copy(dstf_hbm.at[pl.ds(base, CHUNK)], dst_c)
        pltpu.sync_copy(y1_hbm.at[src_c], rows_v)
        pltpu.sync_copy(rows_v, acc_sh.at[dst_c], add=True)

        def cnt_body(i, c2):
            d16 = dst_c[pl.ds(i * 16, 16)]
            plsc.addupdate_scatter(cnt_v, [d16], ones16)
            return c2

        lax.fori_loop(0, vregs_per_chunk, cnt_body, 0)
        return carry

    lax.fori_loop(0, NCHUNK, chunk, 0)

    plsc.subcore_barrier()
    pltpu.sync_copy(acc_sh.at[pl.ds(sid * ROWS_PER_TILE, ROWS_PER_TILE), :],
                    agg_out.at[cid, pl.ds(sid * ROWS_PER_TILE, ROWS_PER_TILE), :])
    pltpu.sync_copy(cnt_v.at[pl.ds(0, N)], cnt_out.at[pl.ds(wid * N, N)])


_layer1_call = pl.kernel(
    _sc_layer1,
    out_type=(jax.ShapeDtypeStruct((NC, NPAD, HID), jnp.float32),
              jax.ShapeDtypeStruct((NW * N,), jnp.float32)),
    mesh=plsc.VectorSubcoreMesh(core_axis_name="c", subcore_axis_name="s",
                                num_cores=NC, num_subcores=NS),
    compiler_params=pltpu.CompilerParams(needs_layout_passes=False, use_tc_tiling_on_sc=False),
    scratch_types=[
        pltpu.VMEM((CHUNK,), jnp.int32),
        pltpu.VMEM((CHUNK,), jnp.int32),
        pltpu.VMEM((CHUNK, HID), jnp.float32),
        pltpu.VMEM((ROWS_PER_TILE, HID), jnp.float32),
        pltpu.VMEM((NPAD,), jnp.float32),
        pltpu.VMEM_SHARED((NPAD, HID), jnp.float32),
    ],
)


def _tc_mid(agg_ref, cntp_ref, xr_ref, b1_ref, w2l_ref, w2r_ref, b2_ref,
            y2_ref, base2_ref, c_ref):
    cnt = jnp.sum(cntp_ref[...], axis=0)
    c = jnp.maximum(cnt, 1.0)
    agg = (agg_ref[0] + agg_ref[1])[:N, :]
    h = jnp.maximum(agg / c[:, None] + xr_ref[...] + b1_ref[...][None, :], 0.0)
    w2l = w2l_ref[...][:, 0]
    w2r = w2r_ref[...][:, 0]
    y2_ref[...] = jnp.sum(h * w2l[None, :], axis=1)
    base2_ref[...] = jnp.sum(h * w2r[None, :], axis=1) + b2_ref[...][0]
    c_ref[...] = c


_mid_call = pl.pallas_call(
    _tc_mid,
    out_shape=(jax.ShapeDtypeStruct((N,), jnp.float32),
               jax.ShapeDtypeStruct((N,), jnp.float32),
               jax.ShapeDtypeStruct((N,), jnp.float32)),
)


def _sc_layer2(y2_hbm, srcf_hbm, dstf_hbm, out_hbm, y2_v, srcf_v, dstf_v, acc_v):
    cid = lax.axis_index("c")
    sid = lax.axis_index("s")
    wid = cid * NS + sid

    zeros16 = jnp.zeros((16,), jnp.float32)

    def zb(i, carry):
        acc_v[pl.ds(i * 16, 16)] = zeros16
        return carry

    lax.fori_loop(0, NPAD // 16, zb, 0)

    pltpu.sync_copy(y2_hbm, y2_v)
    pltpu.sync_copy(srcf_hbm.at[pl.ds(wid * EPW_PAD, EPW_PAD)], srcf_v)
    pltpu.sync_copy(dstf_hbm.at[pl.ds(wid * EPW_PAD, EPW_PAD)], dstf_v)

    def step(i, carry):
        s16 = srcf_v[pl.ds(i * 16, 16)]
        d16 = dstf_v[pl.ds(i * 16, 16)]
        vals = plsc.load_gather(y2_v, [s16])
        plsc.addupdate_scatter(acc_v, [d16], vals)
        return carry

    lax.fori_loop(0, EPW_PAD // 16, step, 0)

    pltpu.sync_copy(acc_v.at[pl.ds(0, N)], out_hbm.at[pl.ds(wid * N, N)])


_layer2_call = pl.kernel(
    _sc_layer2,
    out_type=jax.ShapeDtypeStruct((NW * N,), jnp.float32),
    mesh=plsc.VectorSubcoreMesh(core_axis_name="c", subcore_axis_name="s",
                                num_cores=NC, num_subcores=NS),
    compiler_params=pltpu.CompilerParams(needs_layout_passes=False, use_tc_tiling_on_sc=False),
    scratch_types=[
        pltpu.VMEM((N,), jnp.float32),
        pltpu.VMEM((EPW_PAD,), jnp.int32),
        pltpu.VMEM((EPW_PAD,), jnp.int32),
        pltpu.VMEM((NPAD,), jnp.float32),
    ],
)


def _tc_final(aggp_ref, c_ref, base2_ref, out_ref):
    s = jnp.sum(aggp_ref[...], axis=0)
    out_ref[...] = s / c_ref[...] + base2_ref[...]


_final_call = pl.pallas_call(
    _tc_final,
    out_shape=jax.ShapeDtypeStruct((N,), jnp.float32),
)


def kernel(x, edge_index, W1_l, W1_r, b1, W2_l, W2_r, b2):
    ei = edge_index.astype(jnp.int32)
    src = ei[0].reshape(NW, EPW)
    dst = ei[1].reshape(NW, EPW)
    pad = EPW_PAD - EPW
    srcp = jnp.concatenate([src, jnp.zeros((NW, pad), jnp.int32)], axis=1).reshape(-1)
    dstp = jnp.concatenate([dst, jnp.full((NW, pad), N, jnp.int32)], axis=1).reshape(-1)
    y1, xr = _transform_call(x, W1_l, W1_r)
    agg_p, cnt_p = _layer1_call(y1, srcp, dstp)
    y2, base2, c = _mid_call(agg_p, cnt_p.reshape(NW, N), xr, b1, W2_l, W2_r, b2)
    agg2_p = _layer2_call(y2, srcp, dstp)
    return _final_call(agg2_p.reshape(NW, N), c, base2)


# speedup vs baseline: 28.3528x; 1.3518x over previous
import functools

import jax
import jax.numpy as jnp
from jax import lax
from jax.experimental import pallas as pl
from jax.experimental.pallas import tpu as pltpu
from jax.experimental.pallas import tpu_sc as plsc

N = 10000
E = 320000
IN_CH = 128
HID = 16

NC, NS = 2, 16
NW = NC * NS
EPW = E // NW

CHUNK = 2048
NFULL = 4
TAIL = EPW - NFULL * CHUNK

NPAD = 10112
ROWS_PER_TILE = NPAD // NS


def _tc_transform(x_ref, wl_ref, wr_ref, y1_ref, xr_ref):
    xx = x_ref[...]
    y1_ref[...] = lax.dot(xx, wl_ref[...], precision=lax.Precision.HIGHEST,
                          preferred_element_type=jnp.float32)
    xr_ref[...] = lax.dot(xx, wr_ref[...], precision=lax.Precision.HIGHEST,
                          preferred_element_type=jnp.float32)


_transform_call = pl.pallas_call(
    _tc_transform,
    out_shape=(jax.ShapeDtypeStruct((N, HID), jnp.float32),
               jax.ShapeDtypeStruct((N, HID), jnp.float32)),
)


def _sc_layer1(y1_hbm, srcf_hbm, dstf_hbm, agg_out, cnt_out,
               src_c, dst_c, src_t, dst_t, rows_v, zrow_v, cnt_v, y1_sh, acc_sh):
    cid = lax.axis_index("c")
    sid = lax.axis_index("s")
    wid = cid * NS + sid

    @pl.when(sid == 0)
    def _():
        pltpu.sync_copy(y1_hbm, y1_sh)

    zeros16 = jnp.zeros((16,), jnp.float32)

    def zb(i, carry):
        zrow_v[i, :] = zeros16
        cnt_v[pl.ds(i * 16, 16)] = zeros16
        return carry

    lax.fori_loop(0, ROWS_PER_TILE, zb, 0)
    pltpu.sync_copy(zrow_v, acc_sh.at[pl.ds(sid * ROWS_PER_TILE, ROWS_PER_TILE), :])

    plsc.subcore_barrier()

    ones16 = jnp.full((16,), 1.0, jnp.float32)

    def do_chunk(idx_ref_s, idx_ref_d, rows_slice, size, base):
        pltpu.sync_copy(srcf_hbm.at[pl.ds(base, size)], idx_ref_s)
        pltpu.sync_copy(dstf_hbm.at[pl.ds(base, size)], idx_ref_d)
        pltpu.sync_copy(y1_sh.at[idx_ref_s], rows_slice)
        pltpu.sync_copy(rows_slice, acc_sh.at[idx_ref_d], add=True)

        def cnt_body(i, c2):
            d16 = idx_ref_d[pl.ds(i * 16, 16)]
            plsc.addupdate_scatter(cnt_v, [d16], ones16)
            return c2

        lax.fori_loop(0, size // 16, cnt_body, 0)

    def chunk(j, carry):
        do_chunk(src_c, dst_c, rows_v, CHUNK, wid * EPW + j * CHUNK)
        return carry

    lax.fori_loop(0, NFULL, chunk, 0)
    do_chunk(src_t, dst_t, rows_v.at[pl.ds(0, TAIL), :], TAIL,
             wid * EPW + NFULL * CHUNK)

    plsc.subcore_barrier()
    pltpu.sync_copy(acc_sh.at[pl.ds(sid * ROWS_PER_TILE, ROWS_PER_TILE), :],
                    agg_out.at[cid, pl.ds(sid * ROWS_PER_TILE, ROWS_PER_TILE), :])
    pltpu.sync_copy(cnt_v.at[pl.ds(0, N)], cnt_out.at[pl.ds(wid * N, N)])


_layer1_call = pl.kernel(
    _sc_layer1,
    out_type=(jax.ShapeDtypeStruct((NC, NPAD, HID), jnp.float32),
              jax.ShapeDtypeStruct((NW * N,), jnp.float32)),
    mesh=plsc.VectorSubcoreMesh(core_axis_name="c", subcore_axis_name="s",
                                num_cores=NC, num_subcores=NS),
    compiler_params=pltpu.CompilerParams(needs_layout_passes=False, use_tc_tiling_on_sc=False),
    scratch_types=[
        pltpu.VMEM((CHUNK,), jnp.int32),
        pltpu.VMEM((CHUNK,), jnp.int32),
        pltpu.VMEM((TAIL,), jnp.int32),
        pltpu.VMEM((TAIL,), jnp.int32),
        pltpu.VMEM((CHUNK, HID), jnp.float32),
        pltpu.VMEM((ROWS_PER_TILE, HID), jnp.float32),
        pltpu.VMEM((NPAD,), jnp.float32),
        pltpu.VMEM_SHARED((N, HID), jnp.float32),
        pltpu.VMEM_SHARED((NPAD, HID), jnp.float32),
    ],
)


def _tc_mid(agg_ref, cntp_ref, xr_ref, b1_ref, w2l_ref, w2r_ref, b2_ref,
            y2_ref, base2_ref, c_ref):
    cnt = jnp.sum(cntp_ref[...], axis=0)
    c = jnp.maximum(cnt, 1.0)
    agg = (agg_ref[0] + agg_ref[1])[:N, :]
    h = jnp.maximum(agg / c[:, None] + xr_ref[...] + b1_ref[...][None, :], 0.0)
    w2l = w2l_ref[...][:, 0]
    w2r = w2r_ref[...][:, 0]
    y2_ref[...] = jnp.sum(h * w2l[None, :], axis=1)
    base2_ref[...] = jnp.sum(h * w2r[None, :], axis=1) + b2_ref[...][0]
    c_ref[...] = c


_mid_call = pl.pallas_call(
    _tc_mid,
    out_shape=(jax.ShapeDtypeStruct((N,), jnp.float32),
               jax.ShapeDtypeStruct((N,), jnp.float32),
               jax.ShapeDtypeStruct((N,), jnp.float32)),
)


def _sc_layer2(y2_hbm, srcf_hbm, dstf_hbm, out_hbm, y2_v, srcf_v, dstf_v, acc_v):
    cid = lax.axis_index("c")
    sid = lax.axis_index("s")
    wid = cid * NS + sid

    zeros16 = jnp.zeros((16,), jnp.float32)

    def zb(i, carry):
        acc_v[pl.ds(i * 16, 16)] = zeros16
        return carry

    lax.fori_loop(0, N // 16, zb, 0)

    pltpu.sync_copy(y2_hbm, y2_v)
    pltpu.sync_copy(srcf_hbm.at[pl.ds(wid * EPW, EPW)], srcf_v)
    pltpu.sync_copy(dstf_hbm.at[pl.ds(wid * EPW, EPW)], dstf_v)

    def step(i, carry):
        s16 = srcf_v[pl.ds(i * 16, 16)]
        d16 = dstf_v[pl.ds(i * 16, 16)]
        vals = plsc.load_gather(y2_v, [s16])
        plsc.addupdate_scatter(acc_v, [d16], vals)
        return carry

    lax.fori_loop(0, EPW // 16, step, 0)

    pltpu.sync_copy(acc_v.at[pl.ds(0, N)], out_hbm.at[pl.ds(wid * N, N)])


_layer2_call = pl.kernel(
    _sc_layer2,
    out_type=jax.ShapeDtypeStruct((NW * N,), jnp.float32),
    mesh=plsc.VectorSubcoreMesh(core_axis_name="c", subcore_axis_name="s",
                                num_cores=NC, num_subcores=NS),
    compiler_params=pltpu.CompilerParams(needs_layout_passes=False, use_tc_tiling_on_sc=False),
    scratch_types=[
        pltpu.VMEM((N,), jnp.float32),
        pltpu.VMEM((EPW,), jnp.int32),
        pltpu.VMEM((EPW,), jnp.int32),
        pltpu.VMEM((N,), jnp.float32),
    ],
)


def _tc_final(aggp_ref, c_ref, base2_ref, out_ref):
    s = jnp.sum(aggp_ref[...], axis=0)
    out_ref[...] = s / c_ref[...] + base2_ref[...]


_final_call = pl.pallas_call(
    _tc_final,
    out_shape=jax.ShapeDtypeStruct((N,), jnp.float32),
)


def kernel(x, edge_index, W1_l, W1_r, b1, W2_l, W2_r, b2):
    ei = edge_index.astype(jnp.int32)
    src = ei[0]
    dst = ei[1]
    y1, xr = _transform_call(x, W1_l, W1_r)
    agg_p, cnt_p = _layer1_call(y1, src, dst)
    y2, base2, c = _mid_call(agg_p, cnt_p.reshape(NW, N), xr, b1, W2_l, W2_r, b2)
    agg2_p = _layer2_call(y2, src, dst)
    return _final_call(agg2_p.reshape(NW, N), c, base2)
